# Initial kernel scaffold; baseline (speedup 1.0000x reference)
import jax
import jax.numpy as jnp
from jax.experimental import pallas as pl


def kernel(x, edge_index, W0, b0, W1, b1, W2, b2, Wl1, bl1, Wl2, bl2):
    raise NotImplementedError("write your pallas kernel here")


# speedup vs baseline: 19.4465x; 19.4465x over previous
import functools

import jax
import jax.numpy as jnp
from jax import lax
from jax.experimental import pallas as pl
from jax.experimental.pallas import tpu as pltpu
from jax.experimental.pallas import tpu_sc as plsc

N = 10000
E = 320000
D_FEAT = 128
HID = 32
DW = 8

NC = 2
NS = 16
NW = NC * NS
CH = 128
NFULL = E // (CH * NW)
EPW = NFULL * CH
NEXTRA = (E - EPW * NW) // CH
EXTRA_OFF = EPW * NW

RM = 624
REXT = N - NS * RM
EXT_OFF = NS * RM

_mesh = plsc.VectorSubcoreMesh(
    core_axis_name="c", subcore_axis_name="s", num_cores=NC, num_subcores=NS)
_sc_params = pltpu.CompilerParams(use_tc_tiling_on_sc=False)


@functools.partial(
    pl.kernel,
    out_type=jax.ShapeDtypeStruct((NC, N, DW), jnp.float32),
    mesh=_mesh,
    scratch_types=[
        pltpu.VMEM((CH,), jnp.int32),
        pltpu.VMEM((CH, DW), jnp.float32),
        pltpu.VMEM_SHARED((N, DW), jnp.float32),
    ],
    compiler_params=_sc_params,
)
def _deg_kernel(dst_hbm, zeros_hbm, ones_hbm, degp_hbm, idx_v, ones_v,
                deg_sh):
    cid = lax.axis_index("c")
    sid = lax.axis_index("s")
    wid = cid * NS + sid
    row = pl.ds(sid * RM, RM)
    ext = pl.ds(EXT_OFF, REXT)
    pltpu.sync_copy(zeros_hbm.at[row], deg_sh.at[row])

    @pl.when(sid == NS - 1)
    def _():
        pltpu.sync_copy(zeros_hbm.at[ext], deg_sh.at[ext])

    pltpu.sync_copy(ones_hbm, ones_v)
    plsc.subcore_barrier()
    base = wid * EPW

    def body(i, carry):
        off = base + i * CH
        pltpu.sync_copy(dst_hbm.at[pl.ds(off, CH)], idx_v)
        pltpu.sync_copy(ones_v, deg_sh.at[idx_v], add=True)
        return carry

    lax.fori_loop(0, NFULL, body, 0)

    @pl.when(wid < NEXTRA)
    def _():
        off = EXTRA_OFF + wid * CH
        pltpu.sync_copy(dst_hbm.at[pl.ds(off, CH)], idx_v)
        pltpu.sync_copy(ones_v, deg_sh.at[idx_v], add=True)

    plsc.subcore_barrier()
    pltpu.sync_copy(deg_sh.at[row], degp_hbm.at[cid, row])

    @pl.when(sid == NS - 1)
    def _():
        pltpu.sync_copy(deg_sh.at[ext], degp_hbm.at[cid, ext])


@functools.partial(
    pl.kernel,
    out_type=jax.ShapeDtypeStruct((NC, N, HID), jnp.float32),
    mesh=_mesh,
    scratch_types=[
        pltpu.VMEM((CH,), jnp.int32),
        pltpu.VMEM((CH,), jnp.int32),
        pltpu.VMEM((CH, HID), jnp.float32),
        pltpu.VMEM_SHARED((N, HID), jnp.float32),
        pltpu.VMEM_SHARED((N, HID), jnp.float32),
        pltpu.SemaphoreType.DMA,
    ],
    compiler_params=_sc_params,
)
def _msg_kernel(g_hbm, src_hbm, dst_hbm, out_hbm,
                sidx_v, didx_v, rows_v, g_sh, acc_sh, sem):
    cid = lax.axis_index("c")
    sid = lax.axis_index("s")
    wid = cid * NS + sid
    row = pl.ds(sid * RM, RM)
    ext = pl.ds(EXT_OFF, REXT)
    pltpu.sync_copy(g_hbm.at[row], g_sh.at[row])
    pltpu.sync_copy(g_hbm.at[row], acc_sh.at[row])

    @pl.when(sid == NS - 1)
    def _():
        pltpu.sync_copy(g_hbm.at[ext], g_sh.at[ext])
        pltpu.sync_copy(g_hbm.at[ext], acc_sh.at[ext])

    plsc.subcore_barrier()
    base = wid * EPW

    def body(i, carry):
        off = base + i * CH
        pltpu.sync_copy(src_hbm.at[pl.ds(off, CH)], sidx_v)
        pltpu.sync_copy(dst_hbm.at[pl.ds(off, CH)], didx_v)
        pltpu.async_copy(g_sh.at[sidx_v], rows_v, sem).wait()
        pltpu.sync_copy(rows_v, acc_sh.at[didx_v], add=True)
        return carry

    lax.fori_loop(0, NFULL, body, 0)

    @pl.when(wid < NEXTRA)
    def _():
        off = EXTRA_OFF + wid * CH
        pltpu.sync_copy(src_hbm.at[pl.ds(off, CH)], sidx_v)
        pltpu.sync_copy(dst_hbm.at[pl.ds(off, CH)], didx_v)
        pltpu.async_copy(g_sh.at[sidx_v], rows_v, sem).wait()
        pltpu.sync_copy(rows_v, acc_sh.at[didx_v], add=True)

    plsc.subcore_barrier()
    pltpu.sync_copy(acc_sh.at[row], out_hbm.at[cid, row])

    @pl.when(sid == NS - 1)
    def _():
        pltpu.sync_copy(acc_sh.at[ext], out_hbm.at[cid, ext])


R = 2000


def _ab_body(x_ref, w_ref, degp_ref, dinv_ref, g_ref):
    deg = degp_ref[:, 0:1] + degp_ref[:, 1:2] + 1.0
    dinv = lax.rsqrt(deg)
    dinv_ref[...] = dinv
    t = jnp.dot(x_ref[...], w_ref[...], preferred_element_type=jnp.float32)
    g_ref[...] = t * dinv


def _ab_call(x, W0, degpT):
    return pl.pallas_call(
        _ab_body,
        grid=(N // R,),
        in_specs=[
            pl.BlockSpec((R, D_FEAT), lambda i: (i, 0)),
            pl.BlockSpec((D_FEAT, HID), lambda i: (0, 0)),
            pl.BlockSpec((R, NC), lambda i: (i, 0)),
        ],
        out_specs=[
            pl.BlockSpec((R, 1), lambda i: (i, 0)),
            pl.BlockSpec((R, HID), lambda i: (i, 0)),
        ],
        out_shape=[
            jax.ShapeDtypeStruct((N, 1), jnp.float32),
            jax.ShapeDtypeStruct((N, HID), jnp.float32),
        ],
    )(x, W0, degpT)


def _layer_body(p_ref, g_ref, dinv_ref, b_ref, w_ref, gnew_ref):
    dinv = dinv_ref[...]
    s = p_ref[0] + p_ref[1] - g_ref[...]
    h = jnp.maximum(s * dinv + b_ref[...], 0.0)
    t = jnp.dot(h, w_ref[...], preferred_element_type=jnp.float32)
    gnew_ref[...] = t * dinv


def _layer_call(p, g, dinv, b, W):
    return pl.pallas_call(
        _layer_body,
        grid=(N // R,),
        in_specs=[
            pl.BlockSpec((NC, R, HID), lambda i: (0, i, 0)),
            pl.BlockSpec((R, HID), lambda i: (i, 0)),
            pl.BlockSpec((R, 1), lambda i: (i, 0)),
            pl.BlockSpec((1, HID), lambda i: (0, 0)),
            pl.BlockSpec((HID, HID), lambda i: (0, 0)),
        ],
        out_specs=pl.BlockSpec((R, HID), lambda i: (i, 0)),
        out_shape=jax.ShapeDtypeStruct((N, HID), jnp.float32),
    )(p, g, dinv, b, W)


def _head_body(p_ref, g_ref, dinv_ref, b2_ref, wl1_ref, bl1_ref, wl2_ref,
               bl2_ref, out_ref):
    dinv = dinv_ref[...]
    s = p_ref[0] + p_ref[1] - g_ref[...]
    h = jnp.maximum(s * dinv + b2_ref[...], 0.0)
    h = jnp.maximum(
        jnp.dot(h, wl1_ref[...], preferred_element_type=jnp.float32)
        + bl1_ref[...], 0.0)
    out_ref[...] = (
        jnp.dot(h, wl2_ref[...], preferred_element_type=jnp.float32)
        + bl2_ref[...])


def _head_call(p, g, dinv, b2, Wl1, bl1, Wl2, bl2):
    return pl.pallas_call(
        _head_body,
        grid=(N // R,),
        in_specs=[
            pl.BlockSpec((NC, R, HID), lambda i: (0, i, 0)),
            pl.BlockSpec((R, HID), lambda i: (i, 0)),
            pl.BlockSpec((R, 1), lambda i: (i, 0)),
            pl.BlockSpec((1, HID), lambda i: (0, 0)),
            pl.BlockSpec((HID, HID), lambda i: (0, 0)),
            pl.BlockSpec((1, HID), lambda i: (0, 0)),
            pl.BlockSpec((HID, 1), lambda i: (0, 0)),
            pl.BlockSpec((1, 1), lambda i: (0, 0)),
        ],
        out_specs=pl.BlockSpec((R, 1), lambda i: (i, 0)),
        out_shape=jax.ShapeDtypeStruct((N, 1), jnp.float32),
    )(p, g, dinv, b2, Wl1, bl1, Wl2, bl2)


def kernel(x, edge_index, W0, b0, W1, b1, W2, b2, Wl1, bl1, Wl2, bl2):
    src = edge_index[0]
    dst = edge_index[1]
    zeros = jnp.zeros((N, DW), jnp.float32)
    ones = jnp.ones((CH, DW), jnp.float32)
    b0r = b0.reshape(1, HID)
    b1r = b1.reshape(1, HID)
    b2r = b2.reshape(1, HID)
    bl1r = bl1.reshape(1, HID)
    bl2r = bl2.reshape(1, 1)
    Wl2r = Wl2.reshape(HID, 1)

    degp = _deg_kernel(dst, zeros, ones)
    degpT = degp[:, :, 0].T
    dinv, g0 = _ab_call(x, W0, degpT)
    p1 = _msg_kernel(g0, src, dst)
    g1 = _layer_call(p1, g0, dinv, b0r, W1)
    p2 = _msg_kernel(g1, src, dst)
    g2 = _layer_call(p2, g1, dinv, b1r, W2)
    p3 = _msg_kernel(g2, src, dst)
    out = _head_call(p3, g2, dinv, b2r, Wl1, bl1r, Wl2r, bl2r)
    return out


# speedup vs baseline: 31.8271x; 1.6367x over previous
import functools

import jax
import jax.numpy as jnp
from jax import lax
from jax.experimental import pallas as pl
from jax.experimental.pallas import tpu as pltpu
from jax.experimental.pallas import tpu_sc as plsc

N = 10000
E = 320000
D_FEAT = 128
HID = 32
DW = 8

NC = 2
NS = 16
NW = NC * NS
CH = 128
NROWS = E // CH
NFULL = NROWS // NW
NEXTRA = NROWS - NFULL * NW
EXTRA_ROW = NFULL * NW

RM = 624
REXT = N - NS * RM
EXT_OFF = NS * RM

_mesh = plsc.VectorSubcoreMesh(
    core_axis_name="c", subcore_axis_name="s", num_cores=NC, num_subcores=NS)
_sc_params = pltpu.CompilerParams(use_tc_tiling_on_sc=False)


@functools.partial(
    pl.kernel,
    out_type=jax.ShapeDtypeStruct((NC, N, DW), jnp.float32),
    mesh=_mesh,
    scratch_types=[
        pltpu.VMEM((NFULL + 1, CH), jnp.int32),
        pltpu.VMEM((CH, DW), jnp.float32),
        pltpu.VMEM_SHARED((N, DW), jnp.float32),
        pltpu.SemaphoreType.DMA,
        pltpu.SemaphoreType.DMA,
    ],
    compiler_params=_sc_params,
)
def _deg_kernel(dst2_hbm, zeros_hbm, ones_hbm, degp_hbm, didx_all, ones_v,
                deg_sh, sem0, sem1):
    cid = lax.axis_index("c")
    sid = lax.axis_index("s")
    wid = cid * NS + sid
    row = pl.ds(sid * RM, RM)
    ext = pl.ds(EXT_OFF, REXT)
    pltpu.sync_copy(zeros_hbm.at[row], deg_sh.at[row])

    @pl.when(sid == NS - 1)
    def _():
        pltpu.sync_copy(zeros_hbm.at[ext], deg_sh.at[ext])

    pltpu.sync_copy(ones_hbm, ones_v)
    pltpu.sync_copy(dst2_hbm.at[pl.ds(wid * NFULL, NFULL)],
                    didx_all.at[pl.ds(0, NFULL)])

    @pl.when(wid < NEXTRA)
    def _():
        pltpu.sync_copy(dst2_hbm.at[EXTRA_ROW + wid], didx_all.at[NFULL])

    plsc.subcore_barrier()

    def body(j, carry):
        c0 = pltpu.async_copy(ones_v, deg_sh.at[didx_all.at[2 * j]], sem0,
                              add=True)
        c1 = pltpu.async_copy(ones_v, deg_sh.at[didx_all.at[2 * j + 1]], sem1,
                              add=True)
        c0.wait()
        c1.wait()
        return carry

    lax.fori_loop(0, NFULL // 2, body, 0)

    @pl.when(wid < NEXTRA)
    def _():
        pltpu.sync_copy(ones_v, deg_sh.at[didx_all.at[NFULL]], add=True)

    plsc.subcore_barrier()
    pltpu.sync_copy(deg_sh.at[row], degp_hbm.at[cid, row])

    @pl.when(sid == NS - 1)
    def _():
        pltpu.sync_copy(deg_sh.at[ext], degp_hbm.at[cid, ext])


@functools.partial(
    pl.kernel,
    out_type=jax.ShapeDtypeStruct((NC, N, HID), jnp.float32),
    mesh=_mesh,
    scratch_types=[
        pltpu.VMEM((NFULL + 1, CH), jnp.int32),
        pltpu.VMEM((NFULL + 1, CH), jnp.int32),
        pltpu.VMEM((CH, HID), jnp.float32),
        pltpu.VMEM((CH, HID), jnp.float32),
        pltpu.VMEM_SHARED((N, HID), jnp.float32),
        pltpu.VMEM_SHARED((N, HID), jnp.float32),
        pltpu.SemaphoreType.DMA,
        pltpu.SemaphoreType.DMA,
    ],
    compiler_params=_sc_params,
)
def _msg_kernel(g_hbm, src2_hbm, dst2_hbm, out_hbm,
                sidx_all, didx_all, rows0, rows1, g_sh, acc_sh, sem0, sem1):
    cid = lax.axis_index("c")
    sid = lax.axis_index("s")
    wid = cid * NS + sid
    row = pl.ds(sid * RM, RM)
    ext = pl.ds(EXT_OFF, REXT)
    pltpu.sync_copy(g_hbm.at[row], g_sh.at[row])
    pltpu.sync_copy(g_hbm.at[row], acc_sh.at[row])

    @pl.when(sid == NS - 1)
    def _():
        pltpu.sync_copy(g_hbm.at[ext], g_sh.at[ext])
        pltpu.sync_copy(g_hbm.at[ext], acc_sh.at[ext])

    pltpu.sync_copy(src2_hbm.at[pl.ds(wid * NFULL, NFULL)],
                    sidx_all.at[pl.ds(0, NFULL)])
    pltpu.sync_copy(dst2_hbm.at[pl.ds(wid * NFULL, NFULL)],
                    didx_all.at[pl.ds(0, NFULL)])

    @pl.when(wid < NEXTRA)
    def _():
        pltpu.sync_copy(src2_hbm.at[EXTRA_ROW + wid], sidx_all.at[NFULL])
        pltpu.sync_copy(dst2_hbm.at[EXTRA_ROW + wid], didx_all.at[NFULL])

    plsc.subcore_barrier()

    def body(j, carry):
        i0 = 2 * j
        i1 = i0 + 1
        c0 = pltpu.async_copy(g_sh.at[sidx_all.at[i0]], rows0, sem0)
        c1 = pltpu.async_copy(g_sh.at[sidx_all.at[i1]], rows1, sem1)
        c0.wait()
        pltpu.sync_copy(rows0, acc_sh.at[didx_all.at[i0]], add=True)
        c1.wait()
        pltpu.sync_copy(rows1, acc_sh.at[didx_all.at[i1]], add=True)
        return carry

    lax.fori_loop(0, NFULL // 2, body, 0)

    @pl.when(wid < NEXTRA)
    def _():
        pltpu.async_copy(g_sh.at[sidx_all.at[NFULL]], rows0, sem0).wait()
        pltpu.sync_copy(rows0, acc_sh.at[didx_all.at[NFULL]], add=True)

    plsc.subcore_barrier()
    pltpu.sync_copy(acc_sh.at[row], out_hbm.at[cid, row])

    @pl.when(sid == NS - 1)
    def _():
        pltpu.sync_copy(acc_sh.at[ext], out_hbm.at[cid, ext])


R = 2000


def _ab_body(x_ref, w_ref, degp_ref, dinv_ref, g_ref):
    deg = degp_ref[:, 0:1] + degp_ref[:, 1:2] + 1.0
    dinv = lax.rsqrt(deg)
    dinv_ref[...] = dinv
    t = jnp.dot(x_ref[...], w_ref[...], preferred_element_type=jnp.float32)
    g_ref[...] = t * dinv


def _ab_call(x, W0, degpT):
    return pl.pallas_call(
        _ab_body,
        grid=(N // R,),
        in_specs=[
            pl.BlockSpec((R, D_FEAT), lambda i: (i, 0)),
            pl.BlockSpec((D_FEAT, HID), lambda i: (0, 0)),
            pl.BlockSpec((R, NC), lambda i: (i, 0)),
        ],
        out_specs=[
            pl.BlockSpec((R, 1), lambda i: (i, 0)),
            pl.BlockSpec((R, HID), lambda i: (i, 0)),
        ],
        out_shape=[
            jax.ShapeDtypeStruct((N, 1), jnp.float32),
            jax.ShapeDtypeStruct((N, HID), jnp.float32),
        ],
    )(x, W0, degpT)


def _layer_body(p_ref, g_ref, dinv_ref, b_ref, w_ref, gnew_ref):
    dinv = dinv_ref[...]
    s = p_ref[0] + p_ref[1] - g_ref[...]
    h = jnp.maximum(s * dinv + b_ref[...], 0.0)
    t = jnp.dot(h, w_ref[...], preferred_element_type=jnp.float32)
    gnew_ref[...] = t * dinv


def _layer_call(p, g, dinv, b, W):
    return pl.pallas_call(
        _layer_body,
        grid=(N // R,),
        in_specs=[
            pl.BlockSpec((NC, R, HID), lambda i: (0, i, 0)),
            pl.BlockSpec((R, HID), lambda i: (i, 0)),
            pl.BlockSpec((R, 1), lambda i: (i, 0)),
            pl.BlockSpec((1, HID), lambda i: (0, 0)),
            pl.BlockSpec((HID, HID), lambda i: (0, 0)),
        ],
        out_specs=pl.BlockSpec((R, HID), lambda i: (i, 0)),
        out_shape=jax.ShapeDtypeStruct((N, HID), jnp.float32),
    )(p, g, dinv, b, W)


def _head_body(p_ref, g_ref, dinv_ref, b2_ref, wl1_ref, bl1_ref, wl2_ref,
               bl2_ref, out_ref):
    dinv = dinv_ref[...]
    s = p_ref[0] + p_ref[1] - g_ref[...]
    h = jnp.maximum(s * dinv + b2_ref[...], 0.0)
    h = jnp.maximum(
        jnp.dot(h, wl1_ref[...], preferred_element_type=jnp.float32)
        + bl1_ref[...], 0.0)
    out_ref[...] = (
        jnp.dot(h, wl2_ref[...], preferred_element_type=jnp.float32)
        + bl2_ref[...])


def _head_call(p, g, dinv, b2, Wl1, bl1, Wl2, bl2):
    return pl.pallas_call(
        _head_body,
        grid=(N // R,),
        in_specs=[
            pl.BlockSpec((NC, R, HID), lambda i: (0, i, 0)),
            pl.BlockSpec((R, HID), lambda i: (i, 0)),
            pl.BlockSpec((R, 1), lambda i: (i, 0)),
            pl.BlockSpec((1, HID), lambda i: (0, 0)),
            pl.BlockSpec((HID, HID), lambda i: (0, 0)),
            pl.BlockSpec((1, HID), lambda i: (0, 0)),
            pl.BlockSpec((HID, 1), lambda i: (0, 0)),
            pl.BlockSpec((1, 1), lambda i: (0, 0)),
        ],
        out_specs=pl.BlockSpec((R, 1), lambda i: (i, 0)),
        out_shape=jax.ShapeDtypeStruct((N, 1), jnp.float32),
    )(p, g, dinv, b2, Wl1, bl1, Wl2, bl2)


def kernel(x, edge_index, W0, b0, W1, b1, W2, b2, Wl1, bl1, Wl2, bl2):
    src = edge_index[0].reshape(NROWS, CH)
    dst = edge_index[1].reshape(NROWS, CH)
    zeros = jnp.zeros((N, DW), jnp.float32)
    ones = jnp.ones((CH, DW), jnp.float32)
    b0r = b0.reshape(1, HID)
    b1r = b1.reshape(1, HID)
    b2r = b2.reshape(1, HID)
    bl1r = bl1.reshape(1, HID)
    bl2r = bl2.reshape(1, 1)
    Wl2r = Wl2.reshape(HID, 1)

    degp = _deg_kernel(dst, zeros, ones)
    degpT = degp[:, :, 0].T
    dinv, g0 = _ab_call(x, W0, degpT)
    p1 = _msg_kernel(g0, src, dst)
    g1 = _layer_call(p1, g0, dinv, b0r, W1)
    p2 = _msg_kernel(g1, src, dst)
    g2 = _layer_call(p2, g1, dinv, b1r, W2)
    p3 = _msg_kernel(g2, src, dst)
    out = _head_call(p3, g2, dinv, b2r, Wl1, bl1r, Wl2r, bl2r)
    return out
